# baseline (speedup 1.0000x reference)
import functools

import jax
import jax.numpy as jnp
from jax.experimental import pallas as pl
from jax.experimental.pallas import tpu as pltpu

K = 8192
DIM = 32
BT = 256


def _rvq_block(z_ref, cb0_ref, cb1_ref, cb2_ref,
               zhat_ref, i0_ref, i1_ref, i2_ref):
    z = z_ref[...]
    residual = z
    z_hat = jnp.zeros_like(z)
    idx_refs = (i0_ref, i1_ref, i2_ref)
    for cb_ref, i_ref in zip((cb0_ref, cb1_ref, cb2_ref), idx_refs):
        W = cb_ref[...]
        rr = jnp.sum(residual * residual, axis=1, keepdims=True)
        ww = jnp.sum(W * W, axis=1)[None, :]
        mm = jax.lax.dot_general(
            residual, W, (((1,), (1,)), ((), ())),
            preferred_element_type=jnp.float32)
        d2 = rr - 2.0 * mm + ww
        m = jnp.min(d2, axis=1, keepdims=True)
        iota = jax.lax.broadcasted_iota(jnp.int32, d2.shape, 1)
        idx = jnp.min(jnp.where(d2 == m, iota, K), axis=1, keepdims=True)
        onehot = (iota == idx).astype(jnp.float32)
        z_q = jax.lax.dot_general(
            onehot, W, (((1,), (0,)), ((), ())),
            preferred_element_type=jnp.float32)
        quant_st = residual + (z_q - residual)
        z_hat = z_hat + quant_st
        residual = residual - quant_st
        i_ref[...] = idx
    zhat_ref[...] = z_hat


def kernel(z, codebook0, codebook1, codebook2):
    B = z.shape[0]
    grid = (B // BT,)
    cb_spec = pl.BlockSpec((K, DIM), lambda i: (0, 0))
    z_hat, i0, i1, i2 = pl.pallas_call(
        _rvq_block,
        grid=grid,
        in_specs=[
            pl.BlockSpec((BT, DIM), lambda i: (i, 0)),
            cb_spec, cb_spec, cb_spec,
        ],
        out_specs=[
            pl.BlockSpec((BT, DIM), lambda i: (i, 0)),
            pl.BlockSpec((BT, 1), lambda i: (i, 0)),
            pl.BlockSpec((BT, 1), lambda i: (i, 0)),
            pl.BlockSpec((BT, 1), lambda i: (i, 0)),
        ],
        out_shape=[
            jax.ShapeDtypeStruct((B, DIM), jnp.float32),
            jax.ShapeDtypeStruct((B, 1), jnp.int32),
            jax.ShapeDtypeStruct((B, 1), jnp.int32),
            jax.ShapeDtypeStruct((B, 1), jnp.int32),
        ],
        compiler_params=pltpu.CompilerParams(
            dimension_semantics=("parallel",)),
    )(z, codebook0, codebook1, codebook2)
    indices = jnp.concatenate([i0, i1, i2], axis=1)
    return z_hat, indices


# speedup vs baseline: 1.2661x; 1.2661x over previous
import functools

import jax
import jax.numpy as jnp
from jax.experimental import pallas as pl
from jax.experimental.pallas import tpu as pltpu

K = 8192
DIM = 32
BT = 256


def _norms_block(cb0_ref, cb1_ref, cb2_ref, w0_ref, w1_ref, w2_ref):
    for cb_ref, w_ref in ((cb0_ref, w0_ref), (cb1_ref, w1_ref),
                          (cb2_ref, w2_ref)):
        W = cb_ref[...]
        w_ref[...] = jnp.sum(W * W, axis=1, keepdims=True)


def _rvq_block(z_ref, t0_ref, t1_ref, t2_ref, w0_ref, w1_ref, w2_ref,
               zhat_ref, i0_ref, i1_ref, i2_ref):
    z = z_ref[...]
    residual = z
    z_hat = jnp.zeros_like(z)
    for wt_ref, ww_ref, i_ref in ((t0_ref, w0_ref, i0_ref),
                                  (t1_ref, w1_ref, i1_ref),
                                  (t2_ref, w2_ref, i2_ref)):
        Wt2 = wt_ref[...]
        ww = ww_ref[...]
        rr = jnp.sum(residual * residual, axis=1, keepdims=True)
        mm2 = jax.lax.dot_general(
            residual, Wt2, (((1,), (0,)), ((), ())),
            preferred_element_type=jnp.float32)
        d2 = rr - mm2 + ww
        idx = jnp.argmin(d2, axis=1).reshape(BT, 1)
        iota = jax.lax.broadcasted_iota(jnp.int32, d2.shape, 1)
        onehot = (iota == idx).astype(jnp.float32)
        z_q = 0.5 * jax.lax.dot_general(
            onehot, Wt2, (((1,), (1,)), ((), ())),
            preferred_element_type=jnp.float32)
        quant_st = residual + (z_q - residual)
        z_hat = z_hat + quant_st
        residual = residual - quant_st
        i_ref[...] = idx
    zhat_ref[...] = z_hat


def kernel(z, codebook0, codebook1, codebook2):
    B = z.shape[0]
    cb_spec = pl.BlockSpec((K, DIM), lambda: (0, 0))
    ww_col = pl.pallas_call(
        _norms_block,
        in_specs=[cb_spec, cb_spec, cb_spec],
        out_specs=[pl.BlockSpec((K, 1), lambda: (0, 0))] * 3,
        out_shape=[jax.ShapeDtypeStruct((K, 1), jnp.float32)] * 3,
    )(codebook0, codebook1, codebook2)
    wws = [w.reshape(1, K) for w in ww_col]
    wts = [2.0 * cb.T for cb in (codebook0, codebook1, codebook2)]

    grid = (B // BT,)
    wt_spec = pl.BlockSpec((DIM, K), lambda i: (0, 0))
    ww_spec = pl.BlockSpec((1, K), lambda i: (0, 0))
    z_hat, i0, i1, i2 = pl.pallas_call(
        _rvq_block,
        grid=grid,
        in_specs=[pl.BlockSpec((BT, DIM), lambda i: (i, 0)),
                  wt_spec, wt_spec, wt_spec,
                  ww_spec, ww_spec, ww_spec],
        out_specs=[
            pl.BlockSpec((BT, DIM), lambda i: (i, 0)),
            pl.BlockSpec((BT, 1), lambda i: (i, 0)),
            pl.BlockSpec((BT, 1), lambda i: (i, 0)),
            pl.BlockSpec((BT, 1), lambda i: (i, 0)),
        ],
        out_shape=[
            jax.ShapeDtypeStruct((B, DIM), jnp.float32),
            jax.ShapeDtypeStruct((B, 1), jnp.int32),
            jax.ShapeDtypeStruct((B, 1), jnp.int32),
            jax.ShapeDtypeStruct((B, 1), jnp.int32),
        ],
        compiler_params=pltpu.CompilerParams(
            dimension_semantics=("parallel",)),
    )(z, *wts, *wws)
    indices = jnp.concatenate([i0, i1, i2], axis=1)
    return z_hat, indices
